# baseline (speedup 1.0000x reference)
import functools

import jax
import jax.numpy as jnp
from jax import lax
from jax.experimental import pallas as pl
from jax.experimental.pallas import tpu as pltpu
from jax.experimental.pallas import tpu_sc as plsc

NC = 2
NS = 16
LANES = 16
NW = NC * NS

ADD_SCORE = 0.5
EPS = 1e-16


def _i32(v):
    return jnp.int32(v)


def _tc_project(x, w8, b8, npad):
    n, d = x.shape
    bn = 2560
    assert npad % bn == 0

    def body(x_ref, w_ref, b_ref, o_ref, xo_ref):
        acc = lax.dot_general(
            w_ref[...], x_ref[...], (((1,), (1,)), ((), ())),
            preferred_element_type=jnp.float32)
        o_ref[...] = acc + b_ref[...][:, 0:1]
        xo_ref[...] = x_ref[...]

    return pl.pallas_call(
        body,
        grid=(npad // bn,),
        in_specs=[
            pl.BlockSpec((bn, d), lambda i: (i, i * 0)),
            pl.BlockSpec((8, d), lambda i: (i * 0, i * 0)),
            pl.BlockSpec((8, 128), lambda i: (i * 0, i * 0)),
        ],
        out_specs=[
            pl.BlockSpec((8, bn), lambda i: (i * 0, i)),
            pl.BlockSpec((bn, d), lambda i: (i, i * 0)),
        ],
        out_shape=[jax.ShapeDtypeStruct((8, npad), jnp.float32),
                   jax.ShapeDtypeStruct((n, d), jnp.float32)],
    )(x, w8, b8)


def _sc_pass1(ch, npad, pq, src3, dst3, zeros):
    mesh = plsc.VectorSubcoreMesh(
        core_axis_name="c", subcore_axis_name="s",
        num_cores=NC, num_subcores=NS)
    n = pq.shape[1]

    @functools.partial(
        pl.kernel, mesh=mesh,
        compiler_params=pltpu.CompilerParams(needs_layout_passes=False),
        out_type=(jax.ShapeDtypeStruct((2, NW, ch * 128), jnp.float32),
                  jax.ShapeDtypeStruct((4, npad), jnp.float32)),
        scratch_types=[
            pltpu.VMEM((n,), jnp.float32),
            pltpu.VMEM((n,), jnp.float32),
            pltpu.VMEM((n,), jnp.float32),
            pltpu.VMEM((n,), jnp.float32),
            pltpu.VMEM((ch, 128), jnp.int32),
            pltpu.VMEM((ch * 128,), jnp.int32),
            pltpu.VMEM((ch * 128,), jnp.float32),
            pltpu.VMEM((ch * 128,), jnp.float32),
            pltpu.VMEM_SHARED((npad,), jnp.float32),
            pltpu.VMEM_SHARED((npad,), jnp.float32),
            pltpu.SemaphoreType.DMA,
            pltpu.SemaphoreType.DMA,
            pltpu.SemaphoreType.DMA,
        ])
    def kern(pq_hbm, src_hbm, dst_hbm, zeros_hbm, xexp_hbm, den_hbm,
             p0, p1, q0, q1, sv, dvf, xb0, xb1, d0sh, d1sh,
             sem_in, sem_sc, sem_wb):
        c = lax.axis_index("c")
        s = lax.axis_index("s")
        wid = s * _i32(NC) + c
        cp = [
            pltpu.async_copy(src_hbm.at[wid], sv, sem_in),
            pltpu.async_copy(dst_hbm.at[wid], dvf, sem_in),
            pltpu.async_copy(pq_hbm.at[_i32(0)], p0, sem_in),
            pltpu.async_copy(pq_hbm.at[_i32(1)], p1, sem_in),
            pltpu.async_copy(pq_hbm.at[_i32(2)], q0, sem_in),
            pltpu.async_copy(pq_hbm.at[_i32(3)], q1, sem_in),
        ]

        @pl.when(s == 0)
        def _():
            pltpu.sync_copy(zeros_hbm, d0sh)
            pltpu.sync_copy(zeros_hbm, d1sh)

        for desc in cp:
            desc.wait()
        plsc.subcore_barrier()

        def row(j, carry):
            for k in range(128 // LANES):
                sl = pl.ds(k * LANES, LANES)
                svv = sv[j, sl]
                dvv = dvf[pl.ds(j * _i32(128) + _i32(k * LANES), LANES)]
                x0 = jnp.exp(plsc.load_gather(p0, [svv]) +
                             plsc.load_gather(q0, [dvv]))
                x1 = jnp.exp(plsc.load_gather(p1, [svv]) +
                             plsc.load_gather(q1, [dvv]))
                fsl = pl.ds(j * _i32(128) + _i32(k * LANES), LANES)
                xb0[fsl] = x0
                xb1[fsl] = x1
            return carry
        lax.fori_loop(_i32(0), _i32(ch), row, 0)

        sc0 = pltpu.async_copy(xb0, d0sh.at[dvf], sem_sc, add=True)
        sc1 = pltpu.async_copy(xb1, d1sh.at[dvf], sem_sc, add=True)
        wb0 = pltpu.async_copy(xb0, xexp_hbm.at[_i32(0), wid], sem_wb)
        wb1 = pltpu.async_copy(xb1, xexp_hbm.at[_i32(1), wid], sem_wb)
        sc0.wait()
        sc1.wait()
        wb0.wait()
        wb1.wait()
        plsc.subcore_barrier()

        @pl.when(s == 0)
        def _():
            pltpu.sync_copy(d0sh, den_hbm.at[_i32(2) * c])
            pltpu.sync_copy(d1sh, den_hbm.at[_i32(2) * c + _i32(1)])

    return kern(pq, src3, dst3, zeros)


def _sc_pass2(ch, npad, den, xexp, dst3):
    mesh = plsc.VectorSubcoreMesh(
        core_axis_name="c", subcore_axis_name="s",
        num_cores=NC, num_subcores=NS)

    @functools.partial(
        pl.kernel, mesh=mesh,
        compiler_params=pltpu.CompilerParams(needs_layout_passes=False),
        out_type=jax.ShapeDtypeStruct((2, NW, ch * 128), jnp.float32),
        scratch_types=[
            pltpu.VMEM((npad,), jnp.float32),
            pltpu.VMEM((npad,), jnp.float32),
            pltpu.VMEM((npad,), jnp.float32),
            pltpu.VMEM((npad,), jnp.float32),
            pltpu.VMEM((ch, 128), jnp.int32),
            pltpu.VMEM((ch * 128,), jnp.float32),
            pltpu.VMEM((ch * 128,), jnp.float32),
            pltpu.SemaphoreType.DMA,
            pltpu.SemaphoreType.DMA,
        ])
    def kern(den_hbm, xexp_hbm, dst_hbm, out_hbm,
             d0, d1, t0, t1, dv, xb0, xb1, sem_in, sem_wb):
        c = lax.axis_index("c")
        s = lax.axis_index("s")
        wid = s * _i32(NC) + c
        cp = [
            pltpu.async_copy(den_hbm.at[_i32(0)], d0, sem_in),
            pltpu.async_copy(den_hbm.at[_i32(2)], t0, sem_in),
            pltpu.async_copy(den_hbm.at[_i32(1)], d1, sem_in),
            pltpu.async_copy(den_hbm.at[_i32(3)], t1, sem_in),
            pltpu.async_copy(dst_hbm.at[wid], dv, sem_in),
            pltpu.async_copy(xexp_hbm.at[_i32(0), wid], xb0, sem_in),
            pltpu.async_copy(xexp_hbm.at[_i32(1), wid], xb1, sem_in),
        ]
        for desc in cp:
            desc.wait()

        def addloop(j, carry):
            for jj in range(8):
                sl = pl.ds(j * _i32(8 * LANES) + _i32(jj * LANES), LANES)
                d0[sl] = d0[sl] + t0[sl]
                d1[sl] = d1[sl] + t1[sl]
            return carry
        lax.fori_loop(_i32(0), _i32(npad // (8 * LANES)), addloop, 0)

        def row(j, carry):
            for k in range(128 // LANES):
                sl = pl.ds(k * LANES, LANES)
                fsl = pl.ds(j * _i32(128) + _i32(k * LANES), LANES)
                dvv = dv[j, sl]
                g0 = plsc.load_gather(d0, [dvv])
                g1 = plsc.load_gather(d1, [dvv])
                xb0[fsl] = xb0[fsl] / (g0 + EPS) + ADD_SCORE
                xb1[fsl] = xb1[fsl] / (g1 + EPS) + ADD_SCORE
            return carry
        lax.fori_loop(_i32(0), _i32(ch), row, 0)

        wb0 = pltpu.async_copy(xb0, out_hbm.at[_i32(0), wid], sem_wb)
        wb1 = pltpu.async_copy(xb1, out_hbm.at[_i32(1), wid], sem_wb)
        wb0.wait()
        wb1.wait()

    return kern(den, xexp, dst3)


def kernel(x, edge_index, batch, W, b):
    n, d = x.shape
    e = edge_index.shape[1]
    npad = ((n + 1 + 255) // 256) * 256
    ept = ((e + NW * 128 - 1) // (NW * 128)) * 128
    ch = ept // 128
    e_pad = ept * NW

    xf = x.astype(jnp.float32)
    wf = W.astype(jnp.float32)
    bf = b.astype(jnp.float32)
    w8 = (jnp.zeros((8, d), jnp.float32)
          .at[0].set(wf[:d, 0]).at[1].set(wf[:d, 1])
          .at[2].set(wf[d:, 0]).at[3].set(wf[d:, 1]))
    b8 = (jnp.zeros((8, 128), jnp.float32)
          .at[2, :].set(bf[0]).at[3, :].set(bf[1]))
    pq, xcopy = _tc_project(xf, w8, b8, npad)

    pad = jnp.full((e_pad - e,), n, jnp.int32)
    src3 = jnp.concatenate([edge_index[0].astype(jnp.int32), pad]).reshape(
        NW, ch, 128)
    dst3 = jnp.concatenate([edge_index[1].astype(jnp.int32), pad]).reshape(
        NW, ch, 128)
    zeros = jnp.zeros((npad,), jnp.float32)

    xexp, den = _sc_pass1(ch, npad, pq, src3,
                          dst3.reshape(NW, ch * 128), zeros)
    outs = _sc_pass2(ch, npad, den, xexp, dst3)
    edge_scores = outs.reshape(2, e_pad)[:, :e].T
    return (xcopy, edge_index, batch, edge_scores)


# speedup vs baseline: 1.0276x; 1.0276x over previous
import functools

import jax
import jax.numpy as jnp
from jax import lax
from jax.experimental import pallas as pl
from jax.experimental.pallas import tpu as pltpu
from jax.experimental.pallas import tpu_sc as plsc

NC = 2
NS = 16
LANES = 16
NW = NC * NS

ADD_SCORE = 0.5
EPS = 1e-16


def _i32(v):
    return jnp.int32(v)


def _tc_project(x, w8, b8, npad):
    n, d = x.shape
    bn = 2560
    assert npad % bn == 0

    def body(x_ref, w_ref, b_ref, o_ref, xo_ref):
        acc = lax.dot_general(
            w_ref[...], x_ref[...], (((1,), (1,)), ((), ())),
            preferred_element_type=jnp.float32)
        o_ref[...] = acc + b_ref[...][:, 0:1]
        xo_ref[...] = x_ref[...]

    return pl.pallas_call(
        body,
        grid=(npad // bn,),
        in_specs=[
            pl.BlockSpec((bn, d), lambda i: (i, i * 0)),
            pl.BlockSpec((8, d), lambda i: (i * 0, i * 0)),
            pl.BlockSpec((8, 128), lambda i: (i * 0, i * 0)),
        ],
        out_specs=[
            pl.BlockSpec((8, bn), lambda i: (i * 0, i)),
            pl.BlockSpec((bn, d), lambda i: (i, i * 0)),
        ],
        out_shape=[jax.ShapeDtypeStruct((8, npad), jnp.float32),
                   jax.ShapeDtypeStruct((n, d), jnp.float32)],
    )(x, w8, b8)


def _sc_pass1(ch, npad, pq, src3, dst3, zeros):
    mesh = plsc.VectorSubcoreMesh(
        core_axis_name="c", subcore_axis_name="s",
        num_cores=NC, num_subcores=NS)
    n = pq.shape[1]

    @functools.partial(
        pl.kernel, mesh=mesh,
        compiler_params=pltpu.CompilerParams(needs_layout_passes=False),
        out_type=(jax.ShapeDtypeStruct((2, NW, ch * 128), jnp.float32),
                  jax.ShapeDtypeStruct((4, npad), jnp.float32)),
        scratch_types=[
            pltpu.VMEM((n,), jnp.float32),
            pltpu.VMEM((n,), jnp.float32),
            pltpu.VMEM((n,), jnp.float32),
            pltpu.VMEM((n,), jnp.float32),
            pltpu.VMEM((ch, 128), jnp.int32),
            pltpu.VMEM((ch * 128,), jnp.int32),
            pltpu.VMEM((ch * 128,), jnp.float32),
            pltpu.VMEM((ch * 128,), jnp.float32),
            pltpu.VMEM_SHARED((npad,), jnp.float32),
            pltpu.VMEM_SHARED((npad,), jnp.float32),
            pltpu.SemaphoreType.DMA,
            pltpu.SemaphoreType.DMA,
            pltpu.SemaphoreType.DMA,
        ])
    def kern(pq_hbm, src_hbm, dst_hbm, zeros_hbm, xexp_hbm, den_hbm,
             p0, p1, q0, q1, sv, dvf, xb0, xb1, d0sh, d1sh,
             sem_in, sem_sc, sem_wb):
        c = lax.axis_index("c")
        s = lax.axis_index("s")
        wid = s * _i32(NC) + c
        cp = [
            pltpu.async_copy(src_hbm.at[wid], sv, sem_in),
            pltpu.async_copy(dst_hbm.at[wid], dvf, sem_in),
            pltpu.async_copy(pq_hbm.at[_i32(0)], p0, sem_in),
            pltpu.async_copy(pq_hbm.at[_i32(1)], p1, sem_in),
            pltpu.async_copy(pq_hbm.at[_i32(2)], q0, sem_in),
            pltpu.async_copy(pq_hbm.at[_i32(3)], q1, sem_in),
        ]

        @pl.when(s == 0)
        def _():
            pltpu.sync_copy(zeros_hbm, d0sh)
            pltpu.sync_copy(zeros_hbm, d1sh)

        for desc in cp:
            desc.wait()
        plsc.subcore_barrier()

        def row(j, carry):
            for k in range(128 // LANES):
                sl = pl.ds(k * LANES, LANES)
                svv = sv[j, sl]
                dvv = dvf[pl.ds(j * _i32(128) + _i32(k * LANES), LANES)]
                x0 = jnp.exp(plsc.load_gather(p0, [svv]) +
                             plsc.load_gather(q0, [dvv]))
                x1 = jnp.exp(plsc.load_gather(p1, [svv]) +
                             plsc.load_gather(q1, [dvv]))
                fsl = pl.ds(j * _i32(128) + _i32(k * LANES), LANES)
                xb0[fsl] = x0
                xb1[fsl] = x1
            return carry
        scs = []
        nchunk = 4
        cb = ch // nchunk
        for blk in range(nchunk):
            lax.fori_loop(_i32(blk * cb), _i32((blk + 1) * cb), row, 0)
            csl = pl.ds(_i32(blk * cb * 128), cb * 128)
            scs.append(pltpu.async_copy(
                xb0.at[csl], d0sh.at[dvf.at[csl]], sem_sc, add=True))
            scs.append(pltpu.async_copy(
                xb1.at[csl], d1sh.at[dvf.at[csl]], sem_sc, add=True))
        wb0 = pltpu.async_copy(xb0, xexp_hbm.at[_i32(0), wid], sem_wb)
        wb1 = pltpu.async_copy(xb1, xexp_hbm.at[_i32(1), wid], sem_wb)
        for desc in scs:
            desc.wait()
        wb0.wait()
        wb1.wait()
        plsc.subcore_barrier()

        @pl.when(s == 0)
        def _():
            pltpu.sync_copy(d0sh, den_hbm.at[_i32(2) * c])
            pltpu.sync_copy(d1sh, den_hbm.at[_i32(2) * c + _i32(1)])

    return kern(pq, src3, dst3, zeros)


def _sc_pass2(ch, npad, den, xexp, dst3):
    mesh = plsc.VectorSubcoreMesh(
        core_axis_name="c", subcore_axis_name="s",
        num_cores=NC, num_subcores=NS)

    @functools.partial(
        pl.kernel, mesh=mesh,
        compiler_params=pltpu.CompilerParams(needs_layout_passes=False),
        out_type=jax.ShapeDtypeStruct((2, NW, ch * 128), jnp.float32),
        scratch_types=[
            pltpu.VMEM((npad,), jnp.float32),
            pltpu.VMEM((npad,), jnp.float32),
            pltpu.VMEM((npad,), jnp.float32),
            pltpu.VMEM((npad,), jnp.float32),
            pltpu.VMEM((ch, 128), jnp.int32),
            pltpu.VMEM((ch * 128,), jnp.float32),
            pltpu.VMEM((ch * 128,), jnp.float32),
            pltpu.SemaphoreType.DMA,
            pltpu.SemaphoreType.DMA,
        ])
    def kern(den_hbm, xexp_hbm, dst_hbm, out_hbm,
             d0, d1, t0, t1, dv, xb0, xb1, sem_in, sem_wb):
        c = lax.axis_index("c")
        s = lax.axis_index("s")
        wid = s * _i32(NC) + c
        cp = [
            pltpu.async_copy(den_hbm.at[_i32(0)], d0, sem_in),
            pltpu.async_copy(den_hbm.at[_i32(2)], t0, sem_in),
            pltpu.async_copy(den_hbm.at[_i32(1)], d1, sem_in),
            pltpu.async_copy(den_hbm.at[_i32(3)], t1, sem_in),
            pltpu.async_copy(dst_hbm.at[wid], dv, sem_in),
            pltpu.async_copy(xexp_hbm.at[_i32(0), wid], xb0, sem_in),
            pltpu.async_copy(xexp_hbm.at[_i32(1), wid], xb1, sem_in),
        ]
        for desc in cp:
            desc.wait()

        def addloop(j, carry):
            for jj in range(8):
                sl = pl.ds(j * _i32(8 * LANES) + _i32(jj * LANES), LANES)
                d0[sl] = d0[sl] + t0[sl]
                d1[sl] = d1[sl] + t1[sl]
            return carry
        lax.fori_loop(_i32(0), _i32(npad // (8 * LANES)), addloop, 0)

        def row(j, carry):
            for k in range(128 // LANES):
                sl = pl.ds(k * LANES, LANES)
                fsl = pl.ds(j * _i32(128) + _i32(k * LANES), LANES)
                dvv = dv[j, sl]
                g0 = plsc.load_gather(d0, [dvv])
                g1 = plsc.load_gather(d1, [dvv])
                xb0[fsl] = xb0[fsl] / (g0 + EPS) + ADD_SCORE
                xb1[fsl] = xb1[fsl] / (g1 + EPS) + ADD_SCORE
            return carry
        lax.fori_loop(_i32(0), _i32(ch), row, 0)

        wb0 = pltpu.async_copy(xb0, out_hbm.at[_i32(0), wid], sem_wb)
        wb1 = pltpu.async_copy(xb1, out_hbm.at[_i32(1), wid], sem_wb)
        wb0.wait()
        wb1.wait()

    return kern(den, xexp, dst3)


def kernel(x, edge_index, batch, W, b):
    n, d = x.shape
    e = edge_index.shape[1]
    npad = ((n + 1 + 255) // 256) * 256
    ept = ((e + NW * 128 - 1) // (NW * 128)) * 128
    ch = ept // 128
    e_pad = ept * NW

    xf = x.astype(jnp.float32)
    wf = W.astype(jnp.float32)
    bf = b.astype(jnp.float32)
    w8 = (jnp.zeros((8, d), jnp.float32)
          .at[0].set(wf[:d, 0]).at[1].set(wf[:d, 1])
          .at[2].set(wf[d:, 0]).at[3].set(wf[d:, 1]))
    b8 = (jnp.zeros((8, 128), jnp.float32)
          .at[2, :].set(bf[0]).at[3, :].set(bf[1]))
    pq, xcopy = _tc_project(xf, w8, b8, npad)

    pad = jnp.full((e_pad - e,), n, jnp.int32)
    src3 = jnp.concatenate([edge_index[0].astype(jnp.int32), pad]).reshape(
        NW, ch, 128)
    dst3 = jnp.concatenate([edge_index[1].astype(jnp.int32), pad]).reshape(
        NW, ch, 128)
    zeros = jnp.zeros((npad,), jnp.float32)

    xexp, den = _sc_pass1(ch, npad, pq, src3,
                          dst3.reshape(NW, ch * 128), zeros)
    outs = _sc_pass2(ch, npad, den, xexp, dst3)
    edge_scores = outs.reshape(2, e_pad)[:, :e].T
    return (xcopy, edge_index, batch, edge_scores)


# speedup vs baseline: 1.0514x; 1.0231x over previous
import functools

import jax
import jax.numpy as jnp
from jax import lax
from jax.experimental import pallas as pl
from jax.experimental.pallas import tpu as pltpu
from jax.experimental.pallas import tpu_sc as plsc

NC = 2
NS = 16
LANES = 16
NW = NC * NS

ADD_SCORE = 0.5
EPS = 1e-16


def _i32(v):
    return jnp.int32(v)


def _tc_project(x, w8, b8, npad):
    n, d = x.shape
    bn = 2560
    assert npad % bn == 0

    def body(x_ref, w_ref, b_ref, o_ref, xo_ref):
        acc = lax.dot_general(
            w_ref[...], x_ref[...], (((1,), (1,)), ((), ())),
            preferred_element_type=jnp.float32)
        o_ref[...] = acc + b_ref[...][:, 0:1]
        xo_ref[...] = x_ref[...]

    return pl.pallas_call(
        body,
        grid=(npad // bn,),
        in_specs=[
            pl.BlockSpec((bn, d), lambda i: (i, i * 0)),
            pl.BlockSpec((8, d), lambda i: (i * 0, i * 0)),
            pl.BlockSpec((8, 128), lambda i: (i * 0, i * 0)),
        ],
        out_specs=[
            pl.BlockSpec((8, bn), lambda i: (i * 0, i)),
            pl.BlockSpec((bn, d), lambda i: (i, i * 0)),
        ],
        out_shape=[jax.ShapeDtypeStruct((8, npad), jnp.float32),
                   jax.ShapeDtypeStruct((n, d), jnp.float32)],
    )(x, w8, b8)


def _sc_pass1(ch, npad, pq, src3, dst3, zeros):
    mesh = plsc.VectorSubcoreMesh(
        core_axis_name="c", subcore_axis_name="s",
        num_cores=NC, num_subcores=NS)
    n = pq.shape[1]

    @functools.partial(
        pl.kernel, mesh=mesh,
        compiler_params=pltpu.CompilerParams(needs_layout_passes=False),
        out_type=(jax.ShapeDtypeStruct((2, NW, ch, 128), jnp.float32),
                  jax.ShapeDtypeStruct((4, npad), jnp.float32)),
        scratch_types=[
            pltpu.VMEM((n,), jnp.float32),
            pltpu.VMEM((n,), jnp.float32),
            pltpu.VMEM((n,), jnp.float32),
            pltpu.VMEM((n,), jnp.float32),
            pltpu.VMEM((ch, 128), jnp.int32),
            pltpu.VMEM((ch, 128), jnp.int32),
            pltpu.VMEM((ch, 128), jnp.float32),
            pltpu.VMEM((ch, 128), jnp.float32),
            pltpu.VMEM_SHARED((npad,), jnp.float32),
            pltpu.VMEM_SHARED((npad,), jnp.float32),
            pltpu.SemaphoreType.DMA,
            pltpu.SemaphoreType.DMA,
            pltpu.SemaphoreType.DMA,
        ])
    def kern(pq_hbm, src_hbm, dst_hbm, zeros_hbm, xexp_hbm, den_hbm,
             p0, p1, q0, q1, sv, dv, xb0, xb1, d0sh, d1sh,
             sem_in, sem_sc, sem_wb):
        c = lax.axis_index("c")
        s = lax.axis_index("s")
        wid = s * _i32(NC) + c
        cp = [
            pltpu.async_copy(src_hbm.at[wid], sv, sem_in),
            pltpu.async_copy(dst_hbm.at[wid], dv, sem_in),
            pltpu.async_copy(pq_hbm.at[_i32(0)], p0, sem_in),
            pltpu.async_copy(pq_hbm.at[_i32(1)], p1, sem_in),
            pltpu.async_copy(pq_hbm.at[_i32(2)], q0, sem_in),
            pltpu.async_copy(pq_hbm.at[_i32(3)], q1, sem_in),
        ]

        @pl.when(s == 0)
        def _():
            pltpu.sync_copy(zeros_hbm, d0sh)
            pltpu.sync_copy(zeros_hbm, d1sh)

        for desc in cp:
            desc.wait()
        plsc.subcore_barrier()

        def row(j, carry):
            for k in range(128 // LANES):
                sl = pl.ds(k * LANES, LANES)
                svv = sv[j, sl]
                dvv = dv[j, sl]
                x0 = jnp.exp(plsc.load_gather(p0, [svv]) +
                             plsc.load_gather(q0, [dvv]))
                x1 = jnp.exp(plsc.load_gather(p1, [svv]) +
                             plsc.load_gather(q1, [dvv]))
                xb0[j, sl] = x0
                xb1[j, sl] = x1
            pltpu.async_copy(xb0.at[j], d0sh.at[dv.at[j]], sem_sc, add=True)
            pltpu.async_copy(xb1.at[j], d1sh.at[dv.at[j]], sem_sc, add=True)
            return carry
        lax.fori_loop(_i32(0), _i32(ch), row, 0)

        wb0 = pltpu.async_copy(xb0, xexp_hbm.at[_i32(0), wid], sem_wb)
        wb1 = pltpu.async_copy(xb1, xexp_hbm.at[_i32(1), wid], sem_wb)
        pltpu.make_async_copy(xexp_hbm.at[_i32(0), wid], xb0, sem_sc).wait()
        pltpu.make_async_copy(xexp_hbm.at[_i32(1), wid], xb1, sem_sc).wait()
        wb0.wait()
        wb1.wait()
        plsc.subcore_barrier()

        @pl.when(s == 0)
        def _():
            dp0 = pltpu.async_copy(d0sh, den_hbm.at[_i32(2) * c], sem_wb)
            dp1 = pltpu.async_copy(d1sh, den_hbm.at[_i32(2) * c + _i32(1)],
                                   sem_wb)
            dp0.wait()
            dp1.wait()

    return kern(pq, src3, dst3, zeros)


def _sc_pass2(ch, npad, den, xexp, dst3):
    mesh = plsc.VectorSubcoreMesh(
        core_axis_name="c", subcore_axis_name="s",
        num_cores=NC, num_subcores=NS)

    @functools.partial(
        pl.kernel, mesh=mesh,
        compiler_params=pltpu.CompilerParams(needs_layout_passes=False),
        out_type=jax.ShapeDtypeStruct((2, NW, ch, 128), jnp.float32),
        scratch_types=[
            pltpu.VMEM((npad,), jnp.float32),
            pltpu.VMEM((npad,), jnp.float32),
            pltpu.VMEM((npad,), jnp.float32),
            pltpu.VMEM((npad,), jnp.float32),
            pltpu.VMEM((ch, 128), jnp.int32),
            pltpu.VMEM((ch, 128), jnp.float32),
            pltpu.VMEM((ch, 128), jnp.float32),
            pltpu.SemaphoreType.DMA,
            pltpu.SemaphoreType.DMA,
        ])
    def kern(den_hbm, xexp_hbm, dst_hbm, out_hbm,
             d0, d1, t0, t1, dv, xb0, xb1, sem_in, sem_wb):
        c = lax.axis_index("c")
        s = lax.axis_index("s")
        wid = s * _i32(NC) + c
        cp = [
            pltpu.async_copy(den_hbm.at[_i32(0)], d0, sem_in),
            pltpu.async_copy(den_hbm.at[_i32(2)], t0, sem_in),
            pltpu.async_copy(den_hbm.at[_i32(1)], d1, sem_in),
            pltpu.async_copy(den_hbm.at[_i32(3)], t1, sem_in),
            pltpu.async_copy(dst_hbm.at[wid], dv, sem_in),
            pltpu.async_copy(xexp_hbm.at[_i32(0), wid], xb0, sem_in),
            pltpu.async_copy(xexp_hbm.at[_i32(1), wid], xb1, sem_in),
        ]
        for desc in cp:
            desc.wait()

        def addloop(j, carry):
            for jj in range(8):
                sl = pl.ds(j * _i32(8 * LANES) + _i32(jj * LANES), LANES)
                d0[sl] = d0[sl] + t0[sl]
                d1[sl] = d1[sl] + t1[sl]
            return carry
        lax.fori_loop(_i32(0), _i32(npad // (8 * LANES)), addloop, 0)

        def row(j, carry):
            for k in range(128 // LANES):
                sl = pl.ds(k * LANES, LANES)
                dvv = dv[j, sl]
                g0 = plsc.load_gather(d0, [dvv])
                g1 = plsc.load_gather(d1, [dvv])
                xb0[j, sl] = xb0[j, sl] / (g0 + EPS) + ADD_SCORE
                xb1[j, sl] = xb1[j, sl] / (g1 + EPS) + ADD_SCORE
            return carry
        lax.fori_loop(_i32(0), _i32(ch), row, 0)

        wb0 = pltpu.async_copy(xb0, out_hbm.at[_i32(0), wid], sem_wb)
        wb1 = pltpu.async_copy(xb1, out_hbm.at[_i32(1), wid], sem_wb)
        wb0.wait()
        wb1.wait()

    return kern(den, xexp, dst3)


def kernel(x, edge_index, batch, W, b):
    n, d = x.shape
    e = edge_index.shape[1]
    npad = ((n + 1 + 255) // 256) * 256
    ept = ((e + NW * 128 - 1) // (NW * 128)) * 128
    ch = ept // 128
    e_pad = ept * NW

    xf = x.astype(jnp.float32)
    wf = W.astype(jnp.float32)
    bf = b.astype(jnp.float32)
    w8 = (jnp.zeros((8, d), jnp.float32)
          .at[0].set(wf[:d, 0]).at[1].set(wf[:d, 1])
          .at[2].set(wf[d:, 0]).at[3].set(wf[d:, 1]))
    b8 = (jnp.zeros((8, 128), jnp.float32)
          .at[2, :].set(bf[0]).at[3, :].set(bf[1]))
    pq, xcopy = _tc_project(xf, w8, b8, npad)

    pad = jnp.full((e_pad - e,), n, jnp.int32)
    src3 = jnp.concatenate([edge_index[0].astype(jnp.int32), pad]).reshape(
        NW, ch, 128)
    dst3 = jnp.concatenate([edge_index[1].astype(jnp.int32), pad]).reshape(
        NW, ch, 128)
    zeros = jnp.zeros((npad,), jnp.float32)

    xexp, den = _sc_pass1(ch, npad, pq, src3, dst3, zeros)
    outs = _sc_pass2(ch, npad, den, xexp, dst3)
    edge_scores = outs.reshape(2, e_pad)[:, :e].T
    return (xcopy, edge_index, batch, edge_scores)
